# baseline (speedup 1.0000x reference)
import functools
import jax
import jax.numpy as jnp
import numpy as np
from jax.experimental import pallas as pl
from jax.experimental.pallas import tpu as pltpu

EMB = 64
ROWS_PAD = 50176
BLK = 1568
GRID = ROWS_PAD // BLK
EPS = 1e-5


def _ln_rows(x, g, b):
    mu = jnp.mean(x, axis=-1, keepdims=True)
    var = jnp.mean((x - mu) ** 2, axis=-1, keepdims=True)
    return (x - mu) * jax.lax.rsqrt(var + EPS) * g + b


def _dot(x, w):
    return jax.lax.dot_general(x, w, (((1,), (0,)), ((), ())),
                               preferred_element_type=jnp.float32)


def _proj_body(x_ref, g_ref, b_ref, w1_ref, b1_ref, w2_ref, b2_ref, o_ref):
    x = x_ref[...]
    x = _ln_rows(x, g_ref[...], b_ref[...])
    x = jnp.maximum(_dot(x, w1_ref[...]) + b1_ref[...], 0.0)
    x = jnp.maximum(_dot(x, w2_ref[...]) + b2_ref[...], 0.0)
    o_ref[...] = x


def _proj_tc(x, p):
    f = x.shape[1]
    full = lambda s: pl.BlockSpec(s, lambda i: tuple(0 for _ in s))
    return pl.pallas_call(
        _proj_body,
        grid=(GRID,),
        in_specs=[
            pl.BlockSpec((BLK, f), lambda i: (i, 0)),
            full((f,)), full((f,)),
            full((f, EMB)), full((EMB,)),
            full((EMB, EMB)), full((EMB,)),
        ],
        out_specs=pl.BlockSpec((BLK, EMB), lambda i: (i, 0)),
        out_shape=jax.ShapeDtypeStruct((ROWS_PAD, EMB), jnp.float32),
    )(x, p['ln_g'], p['ln_b'], p['l1']['W'], p['l1']['b'],
      p['l2']['W'], p['l2']['b'])


def _pre_body(r_ref, l_ref, wl_ref, wr_ref, cv_ref, a_ref, b_ref):
    a_ref[...] = _dot(r_ref[...], wl_ref[...]) + cv_ref[...]
    b_ref[...] = _dot(l_ref[...], wr_ref[...])


def _pre_tc(right_t, left_t, wl, wr, cvec):
    full = lambda s: pl.BlockSpec(s, lambda i: tuple(0 for _ in s))
    return pl.pallas_call(
        _pre_body,
        grid=(GRID,),
        in_specs=[
            pl.BlockSpec((BLK, EMB), lambda i: (i, 0)),
            pl.BlockSpec((BLK, EMB), lambda i: (i, 0)),
            full((EMB, EMB)), full((EMB, EMB)), full((EMB,)),
        ],
        out_specs=[pl.BlockSpec((BLK, EMB), lambda i: (i, 0)),
                   pl.BlockSpec((BLK, EMB), lambda i: (i, 0))],
        out_shape=[jax.ShapeDtypeStruct((ROWS_PAD, EMB), jnp.float32),
                   jax.ShapeDtypeStruct((ROWS_PAD, EMB), jnp.float32)],
    )(right_t, left_t, wl, wr, cvec)


def _post_body(s_ref, cnt_ref, r_ref, wf_ref, bf_ref, pg_ref, pb_ref,
               w1a_ref, w1b_ref, b1_ref, w2_ref, b2_ref, o_ref):
    agg = _dot(s_ref[...], wf_ref[...]) + cnt_ref[...] * bf_ref[...]
    h = _ln_rows(agg, pg_ref[...], pb_ref[...])
    z = jnp.maximum(_dot(h, w1a_ref[...]) + _dot(r_ref[...], w1b_ref[...])
                    + b1_ref[...], 0.0)
    o_ref[...] = _dot(z, w2_ref[...]) + b2_ref[...]


def _post_tc(S, cnt, right_t, p):
    full = lambda s: pl.BlockSpec(s, lambda i: tuple(0 for _ in s))
    w1 = p['out1']['W']
    return pl.pallas_call(
        _post_body,
        grid=(GRID,),
        in_specs=[
            pl.BlockSpec((BLK, EMB), lambda i: (i, 0)),
            pl.BlockSpec((BLK, 1), lambda i: (i, 0)),
            pl.BlockSpec((BLK, EMB), lambda i: (i, 0)),
            full((EMB, EMB)), full((EMB,)),
            full((EMB,)), full((EMB,)),
            full((EMB, EMB)), full((EMB, EMB)), full((EMB,)),
            full((EMB, EMB)), full((EMB,)),
        ],
        out_specs=pl.BlockSpec((BLK, EMB), lambda i: (i, 0)),
        out_shape=jax.ShapeDtypeStruct((ROWS_PAD, EMB), jnp.float32),
    )(S, cnt, right_t, p['fin_l']['W'], p['fin_l']['b'],
      p['post_g'], p['post_b'], w1[:EMB], w1[EMB:], p['out1']['b'],
      p['out2']['W'], p['out2']['b'])


def _heads_body(v_ref, c_ref, vw1_ref, vb1_ref, vw2_ref,
                cw1_ref, cb1_ref, cw2_ref, x_ref, lam_ref):
    xv = jnp.maximum(_dot(v_ref[...], vw1_ref[...]) + vb1_ref[...], 0.0)
    x_ref[...] = _dot(xv, vw2_ref[...])
    xc = jnp.maximum(_dot(c_ref[...], cw1_ref[...]) + cb1_ref[...], 0.0)
    t = _dot(xc, cw2_ref[...])
    lam_ref[...] = jnp.maximum(t, 0.0) + jnp.log1p(jnp.exp(-jnp.abs(t)))


def _heads_tc(v_t, c_t, pv, pc):
    full = lambda s: pl.BlockSpec(s, lambda i: tuple(0 for _ in s))
    return pl.pallas_call(
        _heads_body,
        grid=(GRID,),
        in_specs=[
            pl.BlockSpec((BLK, EMB), lambda i: (i, 0)),
            pl.BlockSpec((BLK, EMB), lambda i: (i, 0)),
            full((EMB, EMB)), full((EMB,)), full((EMB, 1)),
            full((EMB, EMB)), full((EMB,)), full((EMB, 1)),
        ],
        out_specs=[pl.BlockSpec((BLK, 1), lambda i: (i, 0)),
                   pl.BlockSpec((BLK, 1), lambda i: (i, 0))],
        out_shape=[jax.ShapeDtypeStruct((ROWS_PAD, 1), jnp.float32),
                   jax.ShapeDtypeStruct((ROWS_PAD, 1), jnp.float32)],
    )(v_t, c_t, pv['l1']['W'], pv['l1']['b'], pv['l2']['W'],
      pc['l1']['W'], pc['l1']['b'], pc['l2']['W'])


def _edge_stage(A, B, src, dst, n_out):
    m = A[dst] + B[src]
    mu = jnp.mean(m, axis=-1, keepdims=True)
    var = jnp.mean((m - mu) ** 2, axis=-1, keepdims=True)
    r = jnp.maximum((m - mu) * jax.lax.rsqrt(var + EPS), 0.0)
    return jax.ops.segment_sum(r, dst, num_segments=n_out)


def _pad_rows(x):
    return jnp.pad(x, ((0, ROWS_PAD - x.shape[0]), (0, 0)))


def kernel(constraint_features, edge_indices, edge_features, variable_features, params):
    del edge_features
    pe = params['edge_proj']
    e1 = jnp.broadcast_to(pe['ln_b'], (1, 1))
    e_vec = jax.nn.relu(jax.nn.relu(e1 @ pe['l1']['W'] + pe['l1']['b'])
                        @ pe['l2']['W'] + pe['l2']['b'])[0]

    cons = _pad_rows(constraint_features)
    var = _pad_rows(variable_features)
    c_t = _proj_tc(cons, params['cons_proj'])
    v_t = _proj_tc(var, params['var_proj'])

    ei = edge_indices.astype(jnp.int32)
    src_c, dst_v = ei[0], ei[1]
    ones = jnp.ones((ei.shape[1],), jnp.float32)
    cnt_c = jax.ops.segment_sum(ones, src_c, num_segments=50000)
    cnt_v = jax.ops.segment_sum(ones, dst_v, num_segments=50000)
    cnt_c = jnp.pad(cnt_c, (0, ROWS_PAD - 50000))[:, None]
    cnt_v = jnp.pad(cnt_v, (0, ROWS_PAD - 50000))[:, None]

    def conv(left_t, right_t, src, dst, cnt, p):
        cvec = p['left']['b'] + e_vec @ p['edge']['W']
        A, B = _pre_tc(right_t, left_t, p['left']['W'], p['right']['W'], cvec)
        g, bb = p['fin_ln_g'], p['fin_ln_b']
        m = A[dst] + B[src]
        mu = jnp.mean(m, axis=-1, keepdims=True)
        vv = jnp.mean((m - mu) ** 2, axis=-1, keepdims=True)
        r = jnp.maximum((m - mu) * jax.lax.rsqrt(vv + EPS) * g + bb, 0.0)
        S = jax.ops.segment_sum(r, dst, num_segments=50000)
        S = jnp.pad(S, ((0, ROWS_PAD - 50000), (0, 0)))
        return _post_tc(S, cnt, right_t, p)

    c_t = conv(v_t, c_t, dst_v, src_c, cnt_c, params['v_to_c'])
    v_t = conv(c_t, v_t, src_c, dst_v, cnt_v, params['c_to_v'])
    c_t = conv(v_t, c_t, dst_v, src_c, cnt_c, params['v_to_c2'])
    v_t = conv(c_t, v_t, src_c, dst_v, cnt_v, params['c_to_v2'])

    x_pad, lam_pad = _heads_tc(v_t, c_t, params['var_head'], params['cons_head'])
    return (x_pad[:50000, 0], lam_pad[:50000, 0])


# speedup vs baseline: 3.1479x; 3.1479x over previous
import functools
import jax
import jax.numpy as jnp
import numpy as np
from jax import lax
from jax.experimental import pallas as pl
from jax.experimental.pallas import tpu as pltpu
from jax.experimental.pallas import tpu_sc as plsc

EMB = 64
N_NODES = 50000
ROWS_PAD = 50176
BLK = 1568
GRID = ROWS_PAD // BLK
EPS = 1e-5

E_PAD = 802816
PAD_IDX = 1 << 20


def _ln_rows(x, g, b):
    mu = jnp.mean(x, axis=-1, keepdims=True)
    var = jnp.mean((x - mu) ** 2, axis=-1, keepdims=True)
    return (x - mu) * jax.lax.rsqrt(var + EPS) * g + b


def _dot(x, w):
    return jax.lax.dot_general(x, w, (((1,), (0,)), ((), ())),
                               preferred_element_type=jnp.float32)


def _proj_body(x_ref, g_ref, b_ref, w1_ref, b1_ref, w2_ref, b2_ref, o_ref):
    x = x_ref[...]
    x = _ln_rows(x, g_ref[...], b_ref[...])
    x = jnp.maximum(_dot(x, w1_ref[...]) + b1_ref[...], 0.0)
    x = jnp.maximum(_dot(x, w2_ref[...]) + b2_ref[...], 0.0)
    o_ref[...] = x


def _proj_tc(x, p):
    f = x.shape[1]
    full = lambda s: pl.BlockSpec(s, lambda i: tuple(0 for _ in s))
    return pl.pallas_call(
        _proj_body,
        grid=(GRID,),
        in_specs=[
            pl.BlockSpec((BLK, f), lambda i: (i, 0)),
            full((f,)), full((f,)),
            full((f, EMB)), full((EMB,)),
            full((EMB, EMB)), full((EMB,)),
        ],
        out_specs=pl.BlockSpec((BLK, EMB), lambda i: (i, 0)),
        out_shape=jax.ShapeDtypeStruct((ROWS_PAD, EMB), jnp.float32),
    )(x, p['ln_g'], p['ln_b'], p['l1']['W'], p['l1']['b'],
      p['l2']['W'], p['l2']['b'])


def _pre_body(r_ref, l_ref, wl_ref, wr_ref, cv_ref, a_ref, b_ref):
    a_ref[...] = _dot(r_ref[...], wl_ref[...]) + cv_ref[...]
    b_ref[...] = _dot(l_ref[...], wr_ref[...])


def _pre_tc(right_t, left_t, wl, wr, cvec):
    full = lambda s: pl.BlockSpec(s, lambda i: tuple(0 for _ in s))
    return pl.pallas_call(
        _pre_body,
        grid=(GRID,),
        in_specs=[
            pl.BlockSpec((BLK, EMB), lambda i: (i, 0)),
            pl.BlockSpec((BLK, EMB), lambda i: (i, 0)),
            full((EMB, EMB)), full((EMB, EMB)), full((EMB,)),
        ],
        out_specs=[pl.BlockSpec((BLK, EMB), lambda i: (i, 0)),
                   pl.BlockSpec((BLK, EMB), lambda i: (i, 0))],
        out_shape=[jax.ShapeDtypeStruct((ROWS_PAD, EMB), jnp.float32),
                   jax.ShapeDtypeStruct((ROWS_PAD, EMB), jnp.float32)],
    )(right_t, left_t, wl, wr, cvec)


def _post_body(s_ref, cnt_ref, r_ref, wf_ref, bf_ref, pg_ref, pb_ref,
               w1a_ref, w1b_ref, b1_ref, w2_ref, b2_ref, o_ref):
    agg = _dot(s_ref[...], wf_ref[...]) + cnt_ref[...] * bf_ref[...]
    h = _ln_rows(agg, pg_ref[...], pb_ref[...])
    z = jnp.maximum(_dot(h, w1a_ref[...]) + _dot(r_ref[...], w1b_ref[...])
                    + b1_ref[...], 0.0)
    o_ref[...] = _dot(z, w2_ref[...]) + b2_ref[...]


def _post_tc(S, cnt, right_t, p):
    full = lambda s: pl.BlockSpec(s, lambda i: tuple(0 for _ in s))
    w1 = p['out1']['W']
    return pl.pallas_call(
        _post_body,
        grid=(GRID,),
        in_specs=[
            pl.BlockSpec((BLK, EMB), lambda i: (i, 0)),
            pl.BlockSpec((BLK, 1), lambda i: (i, 0)),
            pl.BlockSpec((BLK, EMB), lambda i: (i, 0)),
            full((EMB, EMB)), full((EMB,)),
            full((EMB,)), full((EMB,)),
            full((EMB, EMB)), full((EMB, EMB)), full((EMB,)),
            full((EMB, EMB)), full((EMB,)),
        ],
        out_specs=pl.BlockSpec((BLK, EMB), lambda i: (i, 0)),
        out_shape=jax.ShapeDtypeStruct((ROWS_PAD, EMB), jnp.float32),
    )(S, cnt, right_t, p['fin_l']['W'], p['fin_l']['b'],
      p['post_g'], p['post_b'], w1[:EMB], w1[EMB:], p['out1']['b'],
      p['out2']['W'], p['out2']['b'])


def _heads_body(v_ref, c_ref, vw1_ref, vb1_ref, vw2_ref,
                cw1_ref, cb1_ref, cw2_ref, x_ref, lam_ref):
    xv = jnp.maximum(_dot(v_ref[...], vw1_ref[...]) + vb1_ref[...], 0.0)
    x_ref[...] = _dot(xv, vw2_ref[...])
    xc = jnp.maximum(_dot(c_ref[...], cw1_ref[...]) + cb1_ref[...], 0.0)
    t = _dot(xc, cw2_ref[...])
    lam_ref[...] = jnp.maximum(t, 0.0) + jnp.log1p(jnp.exp(-jnp.abs(t)))


def _heads_tc(v_t, c_t, pv, pc):
    full = lambda s: pl.BlockSpec(s, lambda i: tuple(0 for _ in s))
    return pl.pallas_call(
        _heads_body,
        grid=(GRID,),
        in_specs=[
            pl.BlockSpec((BLK, EMB), lambda i: (i, 0)),
            pl.BlockSpec((BLK, EMB), lambda i: (i, 0)),
            full((EMB, EMB)), full((EMB,)), full((EMB, 1)),
            full((EMB, EMB)), full((EMB,)), full((EMB, 1)),
        ],
        out_specs=[pl.BlockSpec((BLK, 1), lambda i: (i, 0)),
                   pl.BlockSpec((BLK, 1), lambda i: (i, 0))],
        out_shape=[jax.ShapeDtypeStruct((ROWS_PAD, 1), jnp.float32),
                   jax.ShapeDtypeStruct((ROWS_PAD, 1), jnp.float32)],
    )(v_t, c_t, pv['l1']['W'], pv['l1']['b'], pv['l2']['W'],
      pc['l1']['W'], pc['l1']['b'], pc['l2']['W'])


_CH = 512
_CHR = _CH // 128
_TPS = 16
_NCH = E_PAD // (_TPS * _CH)
_NRANGE = 8
_Q = 6336
_QP = 6400
_RSTRIPE = _QP // _TPS
_CAP = 2 * _CH


def _rsqrt16(x):
    i = lax.bitcast_convert_type(x, jnp.int32)
    y = lax.bitcast_convert_type(jnp.int32(0x5F3759DF) - (i >> 1), jnp.float32)
    for _ in range(3):
        y = y * (1.5 - 0.5 * x * y * y)
    return y


def _edge_sc(A, B, src2, dst2, g, b):
    mesh = plsc.VectorSubcoreMesh(core_axis_name="c", subcore_axis_name="s")

    @functools.partial(
        pl.kernel,
        out_type=jax.ShapeDtypeStruct((_NRANGE, _QP, EMB), jnp.float32),
        mesh=mesh,
        compiler_params=pltpu.CompilerParams(use_tc_tiling_on_sc=False,
                                            needs_layout_passes=False,
                                            has_side_effects=True),
        scratch_types=[
            pltpu.VMEM((_CH, EMB), jnp.float32),
            pltpu.VMEM((_CH, EMB), jnp.float32),
            pltpu.VMEM((_CH, EMB), jnp.float32),
            pltpu.VMEM((_CHR, 128), jnp.int32),
            pltpu.VMEM((_CHR, 128), jnp.int32),
            pltpu.VMEM((_CAP,), jnp.int32),
            pltpu.VMEM((_CH,), jnp.int32),
            pltpu.VMEM((_CH,), jnp.int32),
            pltpu.VMEM((_CHR, 128), jnp.int32),
            pltpu.VMEM((EMB,), jnp.float32),
            pltpu.VMEM((EMB,), jnp.float32),
            pltpu.VMEM_SHARED((_QP, EMB), jnp.float32),
            pltpu.SemaphoreType.DMA,
        ],
    )
    def k(a_hbm, b_hbm, s_hbm, d_hbm, g_hbm, bb_hbm, out_hbm,
          arows, brows, rrows, sidx, didx, cpk, cdf, csf, ld2d,
          gv, bv, s_sh, sem):
        c = lax.axis_index("c")
        s = lax.axis_index("s")
        pltpu.sync_copy(g_hbm, gv)
        pltpu.sync_copy(bb_hbm, bv)
        gk = [gv[pl.ds(16 * k, 16)] for k in range(4)]
        bk = [bv[pl.ds(16 * k, 16)] for k in range(4)]
        zero16f = jnp.zeros((16,), jnp.float32)
        zero16i = jnp.zeros((16,), jnp.int32)
        iota16 = lax.iota(jnp.int32, 16)
        trash_v = _Q + (iota16 & 3) + 4 * s

        def initf(i, cc):
            cpk[pl.ds(i * 16, 16)] = zero16i
            return cc

        lax.fori_loop(0, _CAP // 16, initf, 0)

        def ebody(e, cc):
            mk = []
            for k in range(4):
                sl = pl.ds(16 * k, 16)
                mk.append(arows[e, sl] + brows[e, sl])
            s1 = (mk[0] + mk[1]) + (mk[2] + mk[3])
            q1 = ((mk[0] * mk[0] + mk[1] * mk[1])
                  + (mk[2] * mk[2] + mk[3] * mk[3]))
            mu_s = jnp.sum(s1) * (1.0 / 64.0)
            var_s = jnp.sum(q1) * (1.0 / 64.0) - mu_s * mu_s + EPS
            y = _rsqrt16(jnp.broadcast_to(var_s, (16,)))
            mu = jnp.broadcast_to(mu_s, (16,))
            for k in range(4):
                t = y * gk[k]
                u = mu * t - bk[k]
                rrows[e, pl.ds(16 * k, 16)] = jnp.maximum(mk[k] * t - u, 0.0)
            return cc

        def process_block(qbase, off_used, trash_v):
            for kk in range(32):
                sl = pl.ds(kk * 16, 16)
                v = cpk[sl]
                d = jnp.minimum(v & 0xFFFF, N_NODES)
                sv = jnp.minimum(lax.shift_right_logical(v, 16), N_NODES)
                cdf[sl] = d
                csf[sl] = sv
                pos = kk * 16 + iota16
                ld2d[kk // 8, pl.ds((kk % 8) * 16, 16)] = jnp.where(
                    pos < off_used, d - qbase, trash_v)
            handles = []
            for j in range(_CHR):
                handles.append(pltpu.async_copy(
                    a_hbm.at[cdf.at[pl.ds(j * 128, 128)]],
                    arows.at[pl.ds(j * 128, 128)], sem))
                handles.append(pltpu.async_copy(
                    b_hbm.at[csf.at[pl.ds(j * 128, 128)]],
                    brows.at[pl.ds(j * 128, 128)], sem))
            for h in handles:
                h.wait()
            lax.fori_loop(0, _CH, ebody, 0)
            for j in range(_CHR):
                pltpu.sync_copy(rrows.at[pl.ds(j * 128, 128)],
                                s_sh.at[ld2d.at[j]], add=True)

        for q in range(_NRANGE // 2):
            qq = c * (_NRANGE // 2) + q
            qbase = qq * _Q
            full_blk = jnp.int32(_CH)

            def zrow(i, cc):
                for k in range(4):
                    rrows[i, pl.ds(16 * k, 16)] = zero16f
                return cc

            lax.fori_loop(0, _CH, zrow, 0)
            done = 0
            while done < _RSTRIPE:
                n = min(_CH, _RSTRIPE - done)
                pltpu.sync_copy(rrows.at[pl.ds(0, n)],
                                s_sh.at[pl.ds(s * _RSTRIPE + done, n)])
                done += n
            plsc.subcore_barrier()

            def chunk(ch, off):
                cid = s * _NCH + ch
                pltpu.sync_copy(s_hbm.at[cid], sidx)
                pltpu.sync_copy(d_hbm.at[cid], didx)
                for j in range(_CHR):
                    for kk in range(8):
                        sl = pl.ds(16 * kk, 16)
                        d = didx[j, sl]
                        sv = sidx[j, sl]
                        l = d - qbase
                        ok = (l >= 0) & (l < _Q)
                        packed = (jnp.minimum(d, 0xFFFF)
                                  | (jnp.minimum(sv, 0xFFFF) << 16))
                        key = jnp.where(ok, 0, 1)
                        _, pv = plsc.sort_key_val(key, packed)
                        cpk[pl.ds(off, 16)] = pv
                        off = off + jnp.sum(jnp.where(ok, 1, 0))

                @pl.when(off >= _CH)
                def _():
                    process_block(qbase, full_blk, trash_v)
                    for kk in range(32):
                        sl = pl.ds(kk * 16, 16)
                        cpk[sl] = cpk[pl.ds(_CH + kk * 16, 16)]

                return jnp.where(off >= _CH, off - _CH, off)

            off = lax.fori_loop(0, _NCH, chunk, 0)
            process_block(qbase, off, trash_v)
            plsc.subcore_barrier()
            pltpu.sync_copy(s_sh.at[pl.ds(s * _RSTRIPE, _RSTRIPE)],
                            out_hbm.at[qq, pl.ds(s * _RSTRIPE, _RSTRIPE)])
            plsc.subcore_barrier()

    return k(A, B, src2, dst2, g, b)


_CNT_R = 25024
_CNT_P = 25088
_CSTRIPE = _CNT_P // _TPS


def _counts_sc(idx2):
    mesh = plsc.VectorSubcoreMesh(core_axis_name="c", subcore_axis_name="s")

    @functools.partial(
        pl.kernel,
        out_type=jax.ShapeDtypeStruct((2, 2, _CNT_P, 16), jnp.float32),
        mesh=mesh,
        compiler_params=pltpu.CompilerParams(use_tc_tiling_on_sc=False,
                                            needs_layout_passes=False,
                                            has_side_effects=True),
        scratch_types=[
            pltpu.VMEM((_CH, 16), jnp.float32),
            pltpu.VMEM((_CH, 16), jnp.float32),
            pltpu.VMEM((_CHR, 128), jnp.int32),
            pltpu.VMEM((_CHR, 128), jnp.int32),
            pltpu.VMEM_SHARED((_CNT_P, 16), jnp.float32),
        ],
    )
    def k(i_hbm, out_hbm, ones_v, zeros_v, ridx, lidx, s_sh):
        c = lax.axis_index("c")
        s = lax.axis_index("s")
        iota16 = lax.iota(jnp.int32, 16)
        one_row = jnp.where(iota16 == 0, 1.0, 0.0).astype(jnp.float32)
        zrow = jnp.zeros((16,), jnp.float32)
        trash_v = _CNT_R + (iota16 & 3) + 4 * s
        base = c * _CNT_R

        def initf(i, cc):
            ones_v[i, pl.ds(0, 16)] = one_row
            zeros_v[i, pl.ds(0, 16)] = zrow
            return cc

        lax.fori_loop(0, _CH, initf, 0)

        for d in range(2):
            for piece in range(3):
                pltpu.sync_copy(zeros_v, s_sh.at[
                    pl.ds(s * _CSTRIPE + piece * _CH, _CH)])
            pltpu.sync_copy(zeros_v.at[pl.ds(0, 32)],
                            s_sh.at[pl.ds(s * _CSTRIPE + 3 * _CH, 32)])
            plsc.subcore_barrier()

            def chunk(ch, carry):
                pltpu.sync_copy(i_hbm.at[d, s * _NCH + ch], ridx)
                for j in range(_CHR):
                    for kk in range(8):
                        sl = pl.ds(16 * kk, 16)
                        l = ridx[j, sl] - base
                        ok = (l >= 0) & (l < _CNT_R)
                        lidx[j, sl] = jnp.where(ok, l, trash_v)
                for j in range(_CHR):
                    pltpu.sync_copy(ones_v.at[pl.ds(j * 128, 128)],
                                    s_sh.at[lidx.at[j]], add=True)
                return carry

            lax.fori_loop(0, _NCH, chunk, 0)
            plsc.subcore_barrier()
            pltpu.sync_copy(s_sh.at[pl.ds(s * _CSTRIPE, _CSTRIPE)],
                            out_hbm.at[d, c, pl.ds(s * _CSTRIPE, _CSTRIPE)])
            plsc.subcore_barrier()

    return k(idx2)


def _pad_rows(x):
    return jnp.pad(x, ((0, ROWS_PAD - x.shape[0]), (0, 0)))


def kernel(constraint_features, edge_indices, edge_features, variable_features, params):
    del edge_features
    pe = params['edge_proj']
    e1 = jnp.broadcast_to(pe['ln_b'], (1, 1))
    e_vec = jax.nn.relu(jax.nn.relu(e1 @ pe['l1']['W'] + pe['l1']['b'])
                        @ pe['l2']['W'] + pe['l2']['b'])[0]

    cons = _pad_rows(constraint_features)
    var = _pad_rows(variable_features)
    c_t = _proj_tc(cons, params['cons_proj'])
    v_t = _proj_tc(var, params['var_proj'])

    ei = edge_indices.astype(jnp.int32)
    ne = ei.shape[1]
    ei_pad = jnp.pad(ei, ((0, 0), (0, E_PAD - ne)), constant_values=PAD_IDX)
    idx2 = ei_pad.reshape(2, E_PAD // _CH, _CHR, 128)

    cnts = _counts_sc(idx2)
    cnt_c = jnp.concatenate([cnts[0, 0, :_CNT_R, :1], cnts[0, 1, :_CNT_R, :1]],
                            axis=0)
    cnt_v = jnp.concatenate([cnts[1, 0, :_CNT_R, :1], cnts[1, 1, :_CNT_R, :1]],
                            axis=0)
    cnt_c = jnp.pad(cnt_c[:N_NODES], ((0, ROWS_PAD - N_NODES), (0, 0)))
    cnt_v = jnp.pad(cnt_v[:N_NODES], ((0, ROWS_PAD - N_NODES), (0, 0)))

    def conv(left_t, right_t, src2, dst2, cnt, p):
        cvec = p['left']['b'] + e_vec @ p['edge']['W']
        A, B = _pre_tc(right_t, left_t, p['left']['W'], p['right']['W'], cvec)
        ranges = _edge_sc(A, B, src2, dst2, p['fin_ln_g'], p['fin_ln_b'])
        S = jnp.concatenate([ranges[i, :_Q] for i in range(_NRANGE)], axis=0)
        return _post_tc(S[:ROWS_PAD], cnt, right_t, p)

    e0, e1 = idx2[0], idx2[1]
    c_t = conv(v_t, c_t, e1, e0, cnt_c, params['v_to_c'])
    v_t = conv(c_t, v_t, e0, e1, cnt_v, params['c_to_v'])
    c_t = conv(v_t, c_t, e1, e0, cnt_c, params['v_to_c2'])
    v_t = conv(c_t, v_t, e0, e1, cnt_v, params['c_to_v2'])

    x_pad, lam_pad = _heads_tc(v_t, c_t, params['var_head'], params['cons_head'])
    return (x_pad[:50000, 0], lam_pad[:50000, 0])
